# baseline (speedup 1.0000x reference)
import functools

import jax
import jax.numpy as jnp
from jax import lax
from jax.experimental import pallas as pl
from jax.experimental.pallas import tpu as pltpu
from jax.experimental.pallas import tpu_sc as plsc

F32 = jnp.float32

N_PAD = 10240
E_PAD = 327680
NW = 32
PW = E_PAD // NW
GCH = 128
NCH = PW // GCH
TPS = N_PAD // 16
BE = 1024
BN = 1024

@functools.cache
def _mesh():
    return plsc.VectorSubcoreMesh(core_axis_name="c", subcore_axis_name="s")


_GK = 4
GGC = 64
NTAB = 10112
TTPS = NTAB // 16


def _sc_gather(xt, row, col):
    he = row.shape[0]
    pw = he // NW
    ngc = pw // GGC

    def body(x_hbm, row_hbm, col_hbm, xr_hbm, xc_hbm, *scr):
        idxv = scr[0]
        xtab = scr[1]
        bufs = scr[2:2 + _GK]
        gs = scr[2 + _GK:2 + 2 * _GK]
        ws = scr[2 + 2 * _GK:2 + 3 * _GK]
        cid = lax.axis_index("c")
        sid = lax.axis_index("s")
        wid = sid * 2 + cid
        base = wid * pw
        pltpu.sync_copy(x_hbm.at[pl.ds(sid * TTPS, TTPS)],
                        xtab.at[pl.ds(sid * TTPS, TTPS)])
        plsc.subcore_barrier()

        def wwait(buf, hbm, sem):
            pltpu.make_async_copy(buf, hbm.at[pl.ds(0, GGC)], sem).wait()

        def phase(idx_hbm, out_hbm):
            pltpu.sync_copy(idx_hbm.at[pl.ds(base, pw)], idxv)

            def step(jj, carry):
                j0 = _GK * jj * GGC

                @pl.when(jj > 0)
                def _drain():
                    for k in range(_GK):
                        wwait(bufs[k], out_hbm, ws[k])

                ds = []
                for k in range(_GK):
                    jk = j0 + k * GGC
                    ds.append(pltpu.async_copy(
                        xtab.at[idxv.at[pl.ds(jk, GGC)]], bufs[k], gs[k]))
                for k in range(_GK):
                    jk = j0 + k * GGC
                    ds[k].wait()
                    pltpu.async_copy(bufs[k],
                                     out_hbm.at[pl.ds(base + jk, GGC)], ws[k])
                return carry

            lax.fori_loop(0, ngc // _GK, step, 0)
            for k in range(_GK):
                wwait(bufs[k], out_hbm, ws[k])

        phase(row_hbm, xr_hbm)
        phase(col_hbm, xc_hbm)

    f = pl.kernel(
        body,
        out_type=(jax.ShapeDtypeStruct((he, 128), F32),
                  jax.ShapeDtypeStruct((he, 128), F32)),
        mesh=_mesh(),
        scratch_types=(
            [pltpu.VMEM((pw,), jnp.int32)]
            + [pltpu.VMEM_SHARED((NTAB, 128), F32)]
            + [pltpu.VMEM((GGC, 128), F32)] * _GK
            + [pltpu.SemaphoreType.DMA] * (2 * _GK)
        ),
    )
    return f(xt, row, col)


def _sc_scatter(eout, col2d, zrows):
    he = eout.shape[0]
    pw = he // NW
    nch = pw // GCH

    def body(eo_hbm, col_hbm, z_hbm, out_hbm, idxv, buf0, buf1, acc,
             ls0, ls1, ss0, ss1):
        cid = lax.axis_index("c")
        sid = lax.axis_index("s")
        wid = sid * 2 + cid
        pltpu.sync_copy(z_hbm, acc.at[pl.ds(sid * TPS, TPS)])
        plsc.subcore_barrier()
        pltpu.sync_copy(col_hbm.at[pl.ds(wid * nch, nch)], idxv)

        def step(jj, carry):
            j0 = 2 * jj
            j1 = j0 + 1
            l0 = pltpu.async_copy(
                eo_hbm.at[pl.ds(wid * pw + j0 * GCH, GCH)], buf0, ls0)
            l1 = pltpu.async_copy(
                eo_hbm.at[pl.ds(wid * pw + j1 * GCH, GCH)], buf1, ls1)
            l0.wait()
            s0 = pltpu.async_copy(buf0, acc.at[idxv.at[j0]], ss0, add=True)
            l1.wait()
            s1 = pltpu.async_copy(buf1, acc.at[idxv.at[j1]], ss1, add=True)
            s0.wait()
            s1.wait()
            return carry

        lax.fori_loop(0, nch // 2, step, 0)
        plsc.subcore_barrier()
        pltpu.sync_copy(acc.at[pl.ds(sid * TPS, TPS)],
                        out_hbm.at[cid, pl.ds(sid * TPS, TPS)])

    f = pl.kernel(
        body,
        out_type=jax.ShapeDtypeStruct((2, N_PAD, 128), F32),
        mesh=_mesh(),
        scratch_types=[
            pltpu.VMEM((nch, GCH), jnp.int32),
            pltpu.VMEM((GCH, 128), F32),
            pltpu.VMEM((GCH, 128), F32),
            pltpu.VMEM_SHARED((N_PAD, 128), F32),
        ] + [pltpu.SemaphoreType.DMA] * 4,
    )
    return f(eout, col2d, zrows)


def _full(shape):
    return pl.BlockSpec(shape, lambda *_: (0,) * len(shape))


def _prelu(h, a):
    return jnp.where(h >= 0, h, a * h)


def _enc_body(x_ref, w1, b1, a, w2, b2, o_ref):
    h = jnp.dot(x_ref[...], w1[...], preferred_element_type=F32) + b1[...]
    h = _prelu(h, a[0, 0])
    o = jnp.dot(h, w2[...], preferred_element_type=F32) + b2[...]
    o_ref[...] = o.astype(o_ref.dtype)


def _tc_encoder(xp, enc, blk, din, out_dtype=F32):
    R = xp.shape[0]
    w1 = enc['l1']['w'].T
    b1 = enc['l1']['b'].reshape(1, 128)
    a = enc['a'].reshape(1, 1)
    w2 = enc['l2']['w'].T
    b2 = enc['l2']['b'].reshape(1, 128)
    return pl.pallas_call(
        _enc_body,
        grid=(R // blk,),
        in_specs=[pl.BlockSpec((blk, din), lambda i: (i, 0)),
                  _full((din, 128)), _full((1, 128)), _full((1, 1)),
                  _full((128, 128)), _full((1, 128))],
        out_specs=pl.BlockSpec((blk, 128), lambda i: (i, 0)),
        out_shape=jax.ShapeDtypeStruct((R, 128), out_dtype),
    )(xp, w1, b1, a, w2, b2)


def _ln_prelu(h, lnw, lnb, a):
    m = jnp.mean(h, axis=1, keepdims=True)
    v = jnp.mean((h - m) ** 2, axis=1, keepdims=True)
    h = (h - m) * lax.rsqrt(v + 1e-5) * lnw + lnb
    return _prelu(h, a)


BF16 = jnp.bfloat16


def _edge_mlp_body(xr_ref, xc_ref, ea_ref, wr, wc, we,
                   b1, lnw, lnb, a, w2, b2, eo_ref, ean_ref):
    ea = ea_ref[...]
    h = (jnp.dot(xr_ref[...], wr[...], preferred_element_type=F32)
         + jnp.dot(xc_ref[...], wc[...], preferred_element_type=F32)
         + jnp.dot(ea, we[...], preferred_element_type=F32)
         + b1[...])
    h = _ln_prelu(h, lnw[...], lnb[...], a[0, 0])
    eo = jnp.dot(h, w2[...], preferred_element_type=F32) + b2[...]
    eo_ref[...] = eo
    ean_ref[...] = ea + eo


def _split_bf(w):
    hi = w.astype(BF16)
    lo = (w - hi.astype(F32)).astype(BF16)
    return hi, lo


def _tc_edge_mlp(xr, xc, ea, m):
    w1 = m['l1']['w']
    wr = w1[:, :128].T
    wc = w1[:, 128:256].T
    we = w1[:, 256:].T
    b1 = m['l1']['b'].reshape(1, 256)
    lnw = m['ln_w'].reshape(1, 256)
    lnb = m['ln_b'].reshape(1, 256)
    a = m['a'].reshape(1, 1)
    w2 = m['l2']['w'].T
    b2 = m['l2']['b'].reshape(1, 128)
    he = xr.shape[0]
    eb = pl.BlockSpec((BE, 128), lambda i: (i, 0))
    return pl.pallas_call(
        _edge_mlp_body,
        grid=(he // BE,),
        in_specs=[eb, eb, eb]
        + [_full((128, 256))] * 3
        + [_full((1, 256))] * 3
        + [_full((1, 1)), _full((256, 128)), _full((1, 128))],
        out_specs=[eb, eb],
        out_shape=[jax.ShapeDtypeStruct((he, 128), F32),
                   jax.ShapeDtypeStruct((he, 128), F32)],
    )(xr, xc, ea, wr, wc, we, b1, lnw, lnb, a, w2, b2)


def _node_mlp_body(*refs):
    (x_ref, wx, wa, b1, lnw, lnb, a, w2, b2, xn_ref) = (
        refs[0], *refs[-9:])
    aggs = refs[1:-9]
    agg = aggs[0][...]
    for r in aggs[1:]:
        agg = agg + r[...]
    h = (jnp.dot(x_ref[...], wx[...], preferred_element_type=F32)
         + jnp.dot(agg, wa[...], preferred_element_type=F32) + b1[...])
    h = _ln_prelu(h, lnw[...], lnb[...], a[0, 0])
    xn_ref[...] = x_ref[...] + jnp.dot(h, w2[...],
                                       preferred_element_type=F32) + b2[...]


def _tc_node_mlp(xp, aggs, m):
    w1 = m['l1']['w']
    wx = w1[:, :128].T
    wa = w1[:, 128:].T
    b1 = m['l1']['b'].reshape(1, 256)
    lnw = m['ln_w'].reshape(1, 256)
    lnb = m['ln_b'].reshape(1, 256)
    a = m['a'].reshape(1, 1)
    w2 = m['l2']['w'].T
    b2 = m['l2']['b'].reshape(1, 128)
    nb = pl.BlockSpec((BN, 128), lambda i: (i, 0))
    return pl.pallas_call(
        _node_mlp_body,
        grid=(N_PAD // BN,),
        in_specs=[nb] * (1 + len(aggs)) +
                 [_full((128, 256)), _full((128, 256)), _full((1, 256)),
                  _full((1, 256)), _full((1, 256)), _full((1, 1)),
                  _full((256, 128)), _full((1, 128))],
        out_specs=nb,
        out_shape=jax.ShapeDtypeStruct((N_PAD, 128), F32),
    )(xp, *aggs, wx, wa, b1, lnw, lnb, a, w2, b2)


def _nodesum_body(x_ref, b_ref, nsT_ref):
    i = pl.program_id(0)

    @pl.when(i == 0)
    def _init():
        nsT_ref[...] = jnp.zeros_like(nsT_ref)

    bv = b_ref[0, 0, :]
    maskT = (bv[:, None] == lax.broadcasted_iota(jnp.int32, (BN, 16), 1))
    nsT_ref[...] += lax.dot_general(
        x_ref[...], maskT.astype(F32), (((0,), (0,)), ((), ())),
        preferred_element_type=F32)


def _tc_nodesum(xp, batch3d):
    return pl.pallas_call(
        _nodesum_body,
        grid=(N_PAD // BN,),
        in_specs=[pl.BlockSpec((BN, 128), lambda i: (i, 0)),
                  pl.BlockSpec((1, 1, BN), lambda i: (i, 0, 0))],
        out_specs=_full((128, 16)),
        out_shape=jax.ShapeDtypeStruct((128, 16), F32),
    )(xp, batch3d)


def _dec_body(globT, w1g, b1g, ag, w2g, b2g, nsT, wd1, wd2, bd, alpha,
              embT, embTs, w1o, b1o, lnwo, lnbo, ao, w2o, b2o, ws1, ws2, bs,
              dos_ref, spark_ref):
    u_preT = _prelu(jnp.dot(w1g[...], globT[...],
                            preferred_element_type=F32) + b1g[...], ag[0, 0])
    uT = jnp.dot(w2g[...], u_preT, preferred_element_type=F32) + b2g[...]
    graphT = (jnp.dot(wd1[...], uT, preferred_element_type=F32)
              + jnp.dot(wd2[...], nsT[...], preferred_element_type=F32)
              + bd[...])
    al = alpha[0, 0]
    eT = embT[...]
    for b in range(16):
        z = eT + al * graphT[:, b:b + 1]
        hT = jnp.dot(w1o[...], z, preferred_element_type=F32) + b1o[...]
        m = jnp.mean(hT, axis=0, keepdims=True)
        v = jnp.mean((hT - m) ** 2, axis=0, keepdims=True)
        hT = (hT - m) * lax.rsqrt(v + 1e-5) * lnwo[...] + lnbo[...]
        hT = _prelu(hT, ao[0, 0])
        dos_ref[b:b + 1, :] = (jnp.dot(w2o[...], hT,
                                       preferred_element_type=F32) + b2o[0, 0])
    s = (jnp.dot(ws1[...], embTs[...], preferred_element_type=F32)
         + jnp.dot(ws2[...], eT, preferred_element_type=F32) + bs[0, 0])
    spark_ref[...] = 1.0 / (1.0 + jnp.exp(-s))


def _tc_decoder(glob, nsT, params):
    ge = params['glob_enc']
    globT = jnp.zeros((8, 16), F32).at[:2, :].set(glob.T)
    w1g = jnp.zeros((128, 8), F32).at[:, :2].set(ge['l1']['w'])
    b1g = ge['l1']['b'].reshape(128, 1)
    ag = ge['a'].reshape(1, 1)
    w2g = ge['l2']['w']
    b2g = ge['l2']['b'].reshape(128, 1)
    wd = params['dec']['w']
    wd1 = wd[:, :128]
    wd2 = wd[:, 128:]
    bd = params['dec']['b'].reshape(128, 1)
    alpha = params['alpha'].reshape(1, 1)
    emb = params['emb']
    embT = jnp.zeros((128, 208), F32).at[:, :201].set(emb.T)
    embTs = jnp.zeros((128, 208), F32).at[:, :200].set(emb[1:201].T)
    po = params['out']
    w1o = po['l1']['w']
    b1o = po['l1']['b'].reshape(128, 1)
    lnwo = po['ln_w'].reshape(128, 1)
    lnbo = po['ln_b'].reshape(128, 1)
    ao = po['a'].reshape(1, 1)
    w2o = po['l2']['w']
    b2o = po['l2']['b'].reshape(1, 1)
    ws = params['spark']['w']
    ws1 = ws[:, :128]
    ws2 = ws[:, 128:]
    bs = params['spark']['b'].reshape(1, 1)
    shapes = [globT, w1g, b1g, ag, w2g, b2g, nsT, wd1, wd2, bd, alpha,
              embT, embTs, w1o, b1o, lnwo, lnbo, ao, w2o, b2o, ws1, ws2, bs]
    return pl.pallas_call(
        _dec_body,
        in_specs=[_full(t.shape) for t in shapes],
        out_specs=[_full((16, 208)), _full((1, 208))],
        out_shape=[jax.ShapeDtypeStruct((16, 208), F32),
                   jax.ShapeDtypeStruct((1, 208), F32)],
    )(*shapes)


def kernel(x, edge_attr, edge_index, glob, batch, params):
    N, H = x.shape
    E = edge_attr.shape[0]
    B = glob.shape[0]

    NS = 10
    HF = E_PAD // NS
    xp = jnp.concatenate([x, jnp.zeros((N_PAD - N, H), F32)])
    eap = jnp.concatenate([edge_attr, jnp.zeros((E_PAD - E, 16), F32)])
    ipad = jnp.full((E_PAD - E,), N, jnp.int32)
    row = jnp.concatenate([edge_index[0], ipad])
    col = jnp.concatenate([edge_index[1], ipad])
    rows = [row[i * HF:(i + 1) * HF] for i in range(NS)]
    cols = [col[i * HF:(i + 1) * HF] for i in range(NS)]
    col2ds = [c.reshape(HF // GCH, GCH) for c in cols]
    batchp = jnp.full((N_PAD,), B, jnp.int32).at[:N].set(batch)
    batch3d = batchp.reshape(N_PAD // BN, 1, BN)
    zrows = jnp.zeros((TPS, 128), F32)

    xcur = _tc_encoder(xp, params['node_enc'], BN, 128)
    eas = [_tc_encoder(eap[i * HF:(i + 1) * HF], params['edge_enc'], BE, 16)
           for i in range(NS)]

    for lp in params['layers']:
        parts = []
        eos = [None] * NS
        for i in range(NS):
            xr, xc = _sc_gather(xcur, rows[i], cols[i])
            if i > 0:
                parts.append(_sc_scatter(eos[i - 1], col2ds[i - 1], zrows))
            eos[i], eas[i] = _tc_edge_mlp(xr, xc, eas[i], lp['edge'])
        parts.append(_sc_scatter(eos[NS - 1], col2ds[NS - 1], zrows))
        aggs = [p[j] for p in parts for j in range(2)]
        xcur = _tc_node_mlp(xcur, aggs, lp['node'])

    nsT = _tc_nodesum(xcur, batch3d)
    dos_p, spark_row = _tc_decoder(glob, nsT, params)

    dos = dos_p[:, :201]
    spark = jnp.broadcast_to(spark_row[0, :200][:, None, None], (200, B, 1))
    return (dos, xcur[:N], spark)


# speedup vs baseline: 1.1064x; 1.1064x over previous
import functools

import jax
import jax.numpy as jnp
from jax import lax
from jax.experimental import pallas as pl
from jax.experimental.pallas import tpu as pltpu
from jax.experimental.pallas import tpu_sc as plsc

F32 = jnp.float32

N_PAD = 10240
E_PAD = 327680
NW = 32
GCH = 128
TPS = N_PAD // 16
BE = 1024
BN = 1024

@functools.cache
def _mesh():
    return plsc.VectorSubcoreMesh(core_axis_name="c", subcore_axis_name="s")


_GK = 4
GGC = 64
NTAB = 10112
TTPS = NTAB // 16


def _sc_gather(xt, row, col):
    he = row.shape[0]
    pw = he // NW
    ngc = pw // GGC

    def body(x_hbm, row_hbm, col_hbm, xr_hbm, xc_hbm, *scr):
        idxv = scr[0]
        xtab = scr[1]
        bufs = scr[2:2 + _GK]
        gs = scr[2 + _GK:2 + 2 * _GK]
        ws = scr[2 + 2 * _GK:2 + 3 * _GK]
        cid = lax.axis_index("c")
        sid = lax.axis_index("s")
        wid = sid * 2 + cid
        base = wid * pw
        pltpu.sync_copy(x_hbm.at[pl.ds(sid * TTPS, TTPS)],
                        xtab.at[pl.ds(sid * TTPS, TTPS)])
        plsc.subcore_barrier()

        def wwait(buf, hbm, sem):
            pltpu.make_async_copy(buf, hbm.at[pl.ds(0, GGC)], sem).wait()

        def phase(idx_hbm, out_hbm):
            pltpu.sync_copy(idx_hbm.at[pl.ds(base, pw)], idxv)

            def step(jj, carry):
                j0 = _GK * jj * GGC

                @pl.when(jj > 0)
                def _drain():
                    for k in range(_GK):
                        wwait(bufs[k], out_hbm, ws[k])

                ds = []
                for k in range(_GK):
                    jk = j0 + k * GGC
                    ds.append(pltpu.async_copy(
                        xtab.at[idxv.at[pl.ds(jk, GGC)]], bufs[k], gs[k]))
                for k in range(_GK):
                    jk = j0 + k * GGC
                    ds[k].wait()
                    pltpu.async_copy(bufs[k],
                                     out_hbm.at[pl.ds(base + jk, GGC)], ws[k])
                return carry

            lax.fori_loop(0, ngc // _GK, step, 0)
            for k in range(_GK):
                wwait(bufs[k], out_hbm, ws[k])

        phase(row_hbm, xr_hbm)
        phase(col_hbm, xc_hbm)

    f = pl.kernel(
        body,
        out_type=(jax.ShapeDtypeStruct((he, 128), F32),
                  jax.ShapeDtypeStruct((he, 128), F32)),
        mesh=_mesh(),
        scratch_types=(
            [pltpu.VMEM((pw,), jnp.int32)]
            + [pltpu.VMEM_SHARED((NTAB, 128), F32)]
            + [pltpu.VMEM((GGC, 128), F32)] * _GK
            + [pltpu.SemaphoreType.DMA] * (2 * _GK)
        ),
    )
    return f(xt, row, col)


def _sc_scatter(eout, col2d, zrows):
    he = eout.shape[0]
    pw = he // NW
    nch = pw // GCH

    def body(eo_hbm, col_hbm, z_hbm, out_hbm, idxv, buf0, buf1, acc,
             ls0, ls1, ss0, ss1):
        cid = lax.axis_index("c")
        sid = lax.axis_index("s")
        wid = sid * 2 + cid
        pltpu.sync_copy(z_hbm, acc.at[pl.ds(sid * TPS, TPS)])
        plsc.subcore_barrier()
        pltpu.sync_copy(col_hbm.at[pl.ds(wid * nch, nch)], idxv)

        def step(jj, carry):
            j0 = 2 * jj
            j1 = j0 + 1
            l0 = pltpu.async_copy(
                eo_hbm.at[pl.ds(wid * pw + j0 * GCH, GCH)], buf0, ls0)
            l1 = pltpu.async_copy(
                eo_hbm.at[pl.ds(wid * pw + j1 * GCH, GCH)], buf1, ls1)
            l0.wait()
            s0 = pltpu.async_copy(buf0, acc.at[idxv.at[j0]], ss0, add=True)
            l1.wait()
            s1 = pltpu.async_copy(buf1, acc.at[idxv.at[j1]], ss1, add=True)
            s0.wait()
            s1.wait()
            return carry

        lax.fori_loop(0, nch // 2, step, 0)
        plsc.subcore_barrier()
        pltpu.sync_copy(acc.at[pl.ds(sid * TPS, TPS)],
                        out_hbm.at[cid, pl.ds(sid * TPS, TPS)])

    f = pl.kernel(
        body,
        out_type=jax.ShapeDtypeStruct((2, N_PAD, 128), F32),
        mesh=_mesh(),
        scratch_types=[
            pltpu.VMEM((nch, GCH), jnp.int32),
            pltpu.VMEM((GCH, 128), F32),
            pltpu.VMEM((GCH, 128), F32),
            pltpu.VMEM_SHARED((N_PAD, 128), F32),
        ] + [pltpu.SemaphoreType.DMA] * 4,
    )
    return f(eout, col2d, zrows)


def _full(shape):
    return pl.BlockSpec(shape, lambda *_: (0,) * len(shape))


def _prelu(h, a):
    return jnp.where(h >= 0, h, a * h)


def _enc_body(x_ref, w1, b1, a, w2, b2, o_ref):
    h = jnp.dot(x_ref[...], w1[...], preferred_element_type=F32) + b1[...]
    h = _prelu(h, a[0, 0])
    o = jnp.dot(h, w2[...], preferred_element_type=F32) + b2[...]
    o_ref[...] = o.astype(o_ref.dtype)


def _tc_encoder(xp, enc, blk, din, out_dtype=F32):
    R = xp.shape[0]
    w1 = enc['l1']['w'].T
    b1 = enc['l1']['b'].reshape(1, 128)
    a = enc['a'].reshape(1, 1)
    w2 = enc['l2']['w'].T
    b2 = enc['l2']['b'].reshape(1, 128)
    return pl.pallas_call(
        _enc_body,
        grid=(R // blk,),
        in_specs=[pl.BlockSpec((blk, din), lambda i: (i, 0)),
                  _full((din, 128)), _full((1, 128)), _full((1, 1)),
                  _full((128, 128)), _full((1, 128))],
        out_specs=pl.BlockSpec((blk, 128), lambda i: (i, 0)),
        out_shape=jax.ShapeDtypeStruct((R, 128), out_dtype),
    )(xp, w1, b1, a, w2, b2)


def _ln_prelu(h, lnw, lnb, a):
    m = jnp.mean(h, axis=1, keepdims=True)
    v = jnp.mean((h - m) ** 2, axis=1, keepdims=True)
    h = (h - m) * lax.rsqrt(v + 1e-5) * lnw + lnb
    return _prelu(h, a)


def _edge_mlp_body(xr_ref, xc_ref, ea_ref, wr, wc, we,
                   b1, lnw, lnb, a, w2, b2, eo_ref, ean_ref):
    ea = ea_ref[...]
    h = (jnp.dot(xr_ref[...], wr[...], preferred_element_type=F32)
         + jnp.dot(xc_ref[...], wc[...], preferred_element_type=F32)
         + jnp.dot(ea, we[...], preferred_element_type=F32)
         + b1[...])
    h = _ln_prelu(h, lnw[...], lnb[...], a[0, 0])
    eo = jnp.dot(h, w2[...], preferred_element_type=F32) + b2[...]
    eo_ref[...] = eo
    ean_ref[...] = ea + eo


def _tc_edge_mlp(xr, xc, ea, m):
    w1 = m['l1']['w']
    wr = w1[:, :128].T
    wc = w1[:, 128:256].T
    we = w1[:, 256:].T
    b1 = m['l1']['b'].reshape(1, 256)
    lnw = m['ln_w'].reshape(1, 256)
    lnb = m['ln_b'].reshape(1, 256)
    a = m['a'].reshape(1, 1)
    w2 = m['l2']['w'].T
    b2 = m['l2']['b'].reshape(1, 128)
    he = xr.shape[0]
    eb = pl.BlockSpec((BE, 128), lambda i: (i, 0))
    return pl.pallas_call(
        _edge_mlp_body,
        grid=(he // BE,),
        in_specs=[eb, eb, eb]
        + [_full((128, 256))] * 3
        + [_full((1, 256))] * 3
        + [_full((1, 1)), _full((256, 128)), _full((1, 128))],
        out_specs=[eb, eb],
        out_shape=[jax.ShapeDtypeStruct((he, 128), F32),
                   jax.ShapeDtypeStruct((he, 128), F32)],
    )(xr, xc, ea, wr, wc, we, b1, lnw, lnb, a, w2, b2)


def _node_mlp_body(*refs):
    (x_ref, wx, wa, b1, lnw, lnb, a, w2, b2, xn_ref) = (
        refs[0], *refs[-9:])
    aggs = refs[1:-9]
    agg = aggs[0][...]
    for r in aggs[1:]:
        agg = agg + r[...]
    h = (jnp.dot(x_ref[...], wx[...], preferred_element_type=F32)
         + jnp.dot(agg, wa[...], preferred_element_type=F32) + b1[...])
    h = _ln_prelu(h, lnw[...], lnb[...], a[0, 0])
    xn_ref[...] = x_ref[...] + jnp.dot(h, w2[...],
                                       preferred_element_type=F32) + b2[...]


def _tc_node_mlp(xp, aggs, m):
    w1 = m['l1']['w']
    wx = w1[:, :128].T
    wa = w1[:, 128:].T
    b1 = m['l1']['b'].reshape(1, 256)
    lnw = m['ln_w'].reshape(1, 256)
    lnb = m['ln_b'].reshape(1, 256)
    a = m['a'].reshape(1, 1)
    w2 = m['l2']['w'].T
    b2 = m['l2']['b'].reshape(1, 128)
    nb = pl.BlockSpec((BN, 128), lambda i: (i, 0))
    return pl.pallas_call(
        _node_mlp_body,
        grid=(N_PAD // BN,),
        in_specs=[nb] * (1 + len(aggs)) +
                 [_full((128, 256)), _full((128, 256)), _full((1, 256)),
                  _full((1, 256)), _full((1, 256)), _full((1, 1)),
                  _full((256, 128)), _full((1, 128))],
        out_specs=nb,
        out_shape=jax.ShapeDtypeStruct((N_PAD, 128), F32),
    )(xp, *aggs, wx, wa, b1, lnw, lnb, a, w2, b2)


def _nodesum_body(x_ref, b_ref, nsT_ref):
    i = pl.program_id(0)

    @pl.when(i == 0)
    def _init():
        nsT_ref[...] = jnp.zeros_like(nsT_ref)

    bv = b_ref[0, 0, :]
    maskT = (bv[:, None] == lax.broadcasted_iota(jnp.int32, (BN, 16), 1))
    nsT_ref[...] += lax.dot_general(
        x_ref[...], maskT.astype(F32), (((0,), (0,)), ((), ())),
        preferred_element_type=F32)


def _tc_nodesum(xp, batch3d):
    return pl.pallas_call(
        _nodesum_body,
        grid=(N_PAD // BN,),
        in_specs=[pl.BlockSpec((BN, 128), lambda i: (i, 0)),
                  pl.BlockSpec((1, 1, BN), lambda i: (i, 0, 0))],
        out_specs=_full((128, 16)),
        out_shape=jax.ShapeDtypeStruct((128, 16), F32),
    )(xp, batch3d)


def _dec_body(globT, w1g, b1g, ag, w2g, b2g, nsT, wd1, wd2, bd, alpha,
              embT, embTs, w1o, b1o, lnwo, lnbo, ao, w2o, b2o, ws1, ws2, bs,
              dos_ref, spark_ref):
    u_preT = _prelu(jnp.dot(w1g[...], globT[...],
                            preferred_element_type=F32) + b1g[...], ag[0, 0])
    uT = jnp.dot(w2g[...], u_preT, preferred_element_type=F32) + b2g[...]
    graphT = (jnp.dot(wd1[...], uT, preferred_element_type=F32)
              + jnp.dot(wd2[...], nsT[...], preferred_element_type=F32)
              + bd[...])
    al = alpha[0, 0]
    eT = embT[...]
    for b in range(16):
        z = eT + al * graphT[:, b:b + 1]
        hT = jnp.dot(w1o[...], z, preferred_element_type=F32) + b1o[...]
        m = jnp.mean(hT, axis=0, keepdims=True)
        v = jnp.mean((hT - m) ** 2, axis=0, keepdims=True)
        hT = (hT - m) * lax.rsqrt(v + 1e-5) * lnwo[...] + lnbo[...]
        hT = _prelu(hT, ao[0, 0])
        dos_ref[b:b + 1, :] = (jnp.dot(w2o[...], hT,
                                       preferred_element_type=F32) + b2o[0, 0])
    s = (jnp.dot(ws1[...], embTs[...], preferred_element_type=F32)
         + jnp.dot(ws2[...], eT, preferred_element_type=F32) + bs[0, 0])
    spark_ref[...] = 1.0 / (1.0 + jnp.exp(-s))


def _tc_decoder(glob, nsT, params):
    ge = params['glob_enc']
    globT = jnp.zeros((8, 16), F32).at[:2, :].set(glob.T)
    w1g = jnp.zeros((128, 8), F32).at[:, :2].set(ge['l1']['w'])
    b1g = ge['l1']['b'].reshape(128, 1)
    ag = ge['a'].reshape(1, 1)
    w2g = ge['l2']['w']
    b2g = ge['l2']['b'].reshape(128, 1)
    wd = params['dec']['w']
    wd1 = wd[:, :128]
    wd2 = wd[:, 128:]
    bd = params['dec']['b'].reshape(128, 1)
    alpha = params['alpha'].reshape(1, 1)
    emb = params['emb']
    embT = jnp.zeros((128, 208), F32).at[:, :201].set(emb.T)
    embTs = jnp.zeros((128, 208), F32).at[:, :200].set(emb[1:201].T)
    po = params['out']
    w1o = po['l1']['w']
    b1o = po['l1']['b'].reshape(128, 1)
    lnwo = po['ln_w'].reshape(128, 1)
    lnbo = po['ln_b'].reshape(128, 1)
    ao = po['a'].reshape(1, 1)
    w2o = po['l2']['w']
    b2o = po['l2']['b'].reshape(1, 1)
    ws = params['spark']['w']
    ws1 = ws[:, :128]
    ws2 = ws[:, 128:]
    bs = params['spark']['b'].reshape(1, 1)
    shapes = [globT, w1g, b1g, ag, w2g, b2g, nsT, wd1, wd2, bd, alpha,
              embT, embTs, w1o, b1o, lnwo, lnbo, ao, w2o, b2o, ws1, ws2, bs]
    return pl.pallas_call(
        _dec_body,
        in_specs=[_full(t.shape) for t in shapes],
        out_specs=[_full((16, 208)), _full((1, 208))],
        out_shape=[jax.ShapeDtypeStruct((16, 208), F32),
                   jax.ShapeDtypeStruct((1, 208), F32)],
    )(*shapes)


def kernel(x, edge_attr, edge_index, glob, batch, params):
    N, H = x.shape
    E = edge_attr.shape[0]
    B = glob.shape[0]

    NS = 5
    HF = E_PAD // NS
    xp = jnp.concatenate([x, jnp.zeros((N_PAD - N, H), F32)])
    eap = jnp.concatenate([edge_attr, jnp.zeros((E_PAD - E, 16), F32)])
    ipad = jnp.full((E_PAD - E,), N, jnp.int32)
    row = jnp.concatenate([edge_index[0], ipad])
    col = jnp.concatenate([edge_index[1], ipad])
    rows = [row[i * HF:(i + 1) * HF] for i in range(NS)]
    cols = [col[i * HF:(i + 1) * HF] for i in range(NS)]
    col2ds = [c.reshape(HF // GCH, GCH) for c in cols]
    batchp = jnp.full((N_PAD,), B, jnp.int32).at[:N].set(batch)
    batch3d = batchp.reshape(N_PAD // BN, 1, BN)
    zrows = jnp.zeros((TPS, 128), F32)

    xcur = _tc_encoder(xp, params['node_enc'], BN, 128)
    eas = [_tc_encoder(eap[i * HF:(i + 1) * HF], params['edge_enc'], BE, 16)
           for i in range(NS)]

    for lp in params['layers']:
        parts = []
        eos = [None] * NS
        for i in range(NS):
            xr, xc = _sc_gather(xcur, rows[i], cols[i])
            if i > 0:
                parts.append(_sc_scatter(eos[i - 1], col2ds[i - 1], zrows))
            eos[i], eas[i] = _tc_edge_mlp(xr, xc, eas[i], lp['edge'])
        parts.append(_sc_scatter(eos[NS - 1], col2ds[NS - 1], zrows))
        aggs = [p[j] for p in parts for j in range(2)]
        xcur = _tc_node_mlp(xcur, aggs, lp['node'])

    nsT = _tc_nodesum(xcur, batch3d)
    dos_p, spark_row = _tc_decoder(glob, nsT, params)

    dos = dos_p[:, :201]
    spark = jnp.broadcast_to(spark_row[0, :200][:, None, None], (200, B, 1))
    return (dos, xcur[:N], spark)
